# baseline (speedup 1.0000x reference)
import functools

import numpy as np
import jax
import jax.numpy as jnp
from jax.experimental import pallas as pl

NUM_NODES = 64
OBS_DIM = 1024
GNN_OBS = OBS_DIM // NUM_NODES
GNN_ACT = 2
HIDDEN = 128
LOG_STD_MIN, LOG_STD_MAX = -5.0, 2.0

BATCH_BLOCK = 8


def _coeffs(rows):
    n = jax.lax.rem(jax.lax.broadcasted_iota(jnp.int32, (rows, 1), 0),
                    NUM_NODES)
    third = jnp.float32(1.0 / 3.0)
    s6 = jnp.float32(1.0 / np.sqrt(6.0))
    last = NUM_NODES - 1
    av = jnp.where((n == 0) | (n == last), jnp.float32(0.5), third)
    lv = jnp.where(n == 0, jnp.float32(0.0),
                   jnp.where((n == 1) | (n == last), s6, third))
    uv = jnp.where(n == last, jnp.float32(0.0),
                   jnp.where((n == 0) | (n == last - 1), s6, third))
    return av, lv, uv


def _agg(h, av, lv, uv):
    prev = jnp.roll(h, 1, axis=0)
    nxt = jnp.roll(h, -1, axis=0)
    return av * h + lv * prev + uv * nxt


def _fused_kernel(x_ref, w0_ref, b0_ref, w1_ref, b1_ref, w2_ref, b2_ref,
                  mu_ref, std_ref):
    rows = x_ref.shape[0]
    av, lv, uv = _coeffs(rows)
    x = x_ref[...]
    h = jnp.dot(x, w0_ref[...], preferred_element_type=jnp.float32)
    h = jax.nn.relu(_agg(h, av, lv, uv) + b0_ref[...])
    h = jnp.dot(h, w1_ref[...], preferred_element_type=jnp.float32)
    h = jax.nn.relu(_agg(h, av, lv, uv) + b1_ref[...])
    h = jnp.dot(h, w2_ref[...], preferred_element_type=jnp.float32)
    h = _agg(h, av, lv, uv) + b2_ref[...]
    mu_ref[...] = h[:, :GNN_ACT]
    ls = jnp.tanh(h[:, GNN_ACT:])
    ls = LOG_STD_MIN + 0.5 * (LOG_STD_MAX - LOG_STD_MIN) * (ls + 1.0)
    std_ref[...] = jnp.exp(ls)


@functools.partial(jax.jit, static_argnames=())
def kernel(obs, W0, b0, W1, b1, W2, b2):
    bs = obs.shape[0]
    rows = bs * NUM_NODES
    block_rows = BATCH_BLOCK * NUM_NODES
    grid = (bs // BATCH_BLOCK,)

    x = obs.reshape(rows, GNN_OBS)

    const = lambda shape: pl.BlockSpec(shape, lambda i: (0, 0))
    mu2, std2 = pl.pallas_call(
        _fused_kernel,
        grid=grid,
        in_specs=[
            pl.BlockSpec((block_rows, GNN_OBS), lambda i: (i, 0)),
            const((GNN_OBS, HIDDEN)),
            const((1, HIDDEN)),
            const((HIDDEN, HIDDEN)),
            const((1, HIDDEN)),
            const((HIDDEN, 2 * GNN_ACT)),
            const((1, 2 * GNN_ACT)),
        ],
        out_specs=[
            pl.BlockSpec((block_rows, GNN_ACT), lambda i: (i, 0)),
            pl.BlockSpec((block_rows, GNN_ACT), lambda i: (i, 0)),
        ],
        out_shape=[
            jax.ShapeDtypeStruct((rows, GNN_ACT), jnp.float32),
            jax.ShapeDtypeStruct((rows, GNN_ACT), jnp.float32),
        ],
    )(x, W0, b0.reshape(1, HIDDEN), W1, b1.reshape(1, HIDDEN),
      W2, b2.reshape(1, 2 * GNN_ACT))

    mu = mu2.reshape(bs, NUM_NODES * GNN_ACT)
    std = std2.reshape(bs, NUM_NODES * GNN_ACT)
    return (mu, std)


# speedup vs baseline: 1.4753x; 1.4753x over previous
import functools

import numpy as np
import jax
import jax.numpy as jnp
from jax.experimental import pallas as pl

NUM_NODES = 64
OBS_DIM = 1024
GNN_OBS = OBS_DIM // NUM_NODES
GNN_ACT = 2
HIDDEN = 128
LOG_STD_MIN, LOG_STD_MAX = -5.0, 2.0

BATCH_BLOCK = 32


def _coeffs(rows):
    n = jax.lax.rem(jax.lax.broadcasted_iota(jnp.int32, (rows, 1), 0),
                    NUM_NODES)
    third = jnp.float32(1.0 / 3.0)
    s6 = jnp.float32(1.0 / np.sqrt(6.0))
    last = NUM_NODES - 1
    av = jnp.where((n == 0) | (n == last), jnp.float32(0.5), third)
    lv = jnp.where(n == 0, jnp.float32(0.0),
                   jnp.where((n == 1) | (n == last), s6, third))
    uv = jnp.where(n == last, jnp.float32(0.0),
                   jnp.where((n == 0) | (n == last - 1), s6, third))
    return av, lv, uv


def _agg(h, av, lv, uv):
    prev = jnp.roll(h, 1, axis=0)
    nxt = jnp.roll(h, -1, axis=0)
    return av * h + lv * prev + uv * nxt


def _fused_kernel(x_ref, w0_ref, b0_ref, w1_ref, b1_ref, w2_ref, b2_ref,
                  mu_ref, std_ref):
    rows = x_ref.shape[0]
    av, lv, uv = _coeffs(rows)
    x = x_ref[...]
    h = jnp.dot(x, w0_ref[...], preferred_element_type=jnp.float32)
    h = jax.nn.relu(_agg(h, av, lv, uv) + b0_ref[...])
    h = jnp.dot(h, w1_ref[...], preferred_element_type=jnp.float32)
    h = jax.nn.relu(_agg(h, av, lv, uv) + b1_ref[...])
    h = jnp.dot(h, w2_ref[...], preferred_element_type=jnp.float32)
    h = _agg(h, av, lv, uv) + b2_ref[...]
    mu_ref[...] = h[:, :GNN_ACT]
    ls = jnp.tanh(h[:, GNN_ACT:])
    ls = LOG_STD_MIN + 0.5 * (LOG_STD_MAX - LOG_STD_MIN) * (ls + 1.0)
    std_ref[...] = jnp.exp(ls)


@functools.partial(jax.jit, static_argnames=())
def kernel(obs, W0, b0, W1, b1, W2, b2):
    bs = obs.shape[0]
    rows = bs * NUM_NODES
    block_rows = BATCH_BLOCK * NUM_NODES
    grid = (bs // BATCH_BLOCK,)

    x = obs.reshape(rows, GNN_OBS)

    const = lambda shape: pl.BlockSpec(shape, lambda i: (0, 0))
    mu2, std2 = pl.pallas_call(
        _fused_kernel,
        grid=grid,
        in_specs=[
            pl.BlockSpec((block_rows, GNN_OBS), lambda i: (i, 0)),
            const((GNN_OBS, HIDDEN)),
            const((1, HIDDEN)),
            const((HIDDEN, HIDDEN)),
            const((1, HIDDEN)),
            const((HIDDEN, 2 * GNN_ACT)),
            const((1, 2 * GNN_ACT)),
        ],
        out_specs=[
            pl.BlockSpec((block_rows, GNN_ACT), lambda i: (i, 0)),
            pl.BlockSpec((block_rows, GNN_ACT), lambda i: (i, 0)),
        ],
        out_shape=[
            jax.ShapeDtypeStruct((rows, GNN_ACT), jnp.float32),
            jax.ShapeDtypeStruct((rows, GNN_ACT), jnp.float32),
        ],
    )(x, W0, b0.reshape(1, HIDDEN), W1, b1.reshape(1, HIDDEN),
      W2, b2.reshape(1, 2 * GNN_ACT))

    mu = mu2.reshape(bs, NUM_NODES * GNN_ACT)
    std = std2.reshape(bs, NUM_NODES * GNN_ACT)
    return (mu, std)


# speedup vs baseline: 1.5742x; 1.0670x over previous
import functools

import numpy as np
import jax
import jax.numpy as jnp
from jax.experimental import pallas as pl

NUM_NODES = 64
OBS_DIM = 1024
GNN_OBS = OBS_DIM // NUM_NODES
GNN_ACT = 2
HIDDEN = 128
LOG_STD_MIN, LOG_STD_MAX = -5.0, 2.0

BATCH_BLOCK = 128


def _coeffs(rows):
    n = jax.lax.rem(jax.lax.broadcasted_iota(jnp.int32, (rows, 1), 0),
                    NUM_NODES)
    third = jnp.float32(1.0 / 3.0)
    s6 = jnp.float32(1.0 / np.sqrt(6.0))
    last = NUM_NODES - 1
    av = jnp.where((n == 0) | (n == last), jnp.float32(0.5), third)
    lv = jnp.where(n == 0, jnp.float32(0.0),
                   jnp.where((n == 1) | (n == last), s6, third))
    uv = jnp.where(n == last, jnp.float32(0.0),
                   jnp.where((n == 0) | (n == last - 1), s6, third))
    return av, lv, uv


def _agg(h, av, lv, uv):
    prev = jnp.roll(h, 1, axis=0)
    nxt = jnp.roll(h, -1, axis=0)
    return av * h + lv * prev + uv * nxt


def _fused_kernel(x_ref, w0_ref, b0_ref, w1_ref, b1_ref, w2_ref, b2_ref,
                  mu_ref, std_ref):
    rows = x_ref.shape[0]
    av, lv, uv = _coeffs(rows)
    x = x_ref[...]
    h = jnp.dot(x, w0_ref[...], preferred_element_type=jnp.float32)
    h = jax.nn.relu(_agg(h, av, lv, uv) + b0_ref[...])
    h = jnp.dot(h, w1_ref[...], preferred_element_type=jnp.float32)
    h = jax.nn.relu(_agg(h, av, lv, uv) + b1_ref[...])
    h = jnp.dot(h, w2_ref[...], preferred_element_type=jnp.float32)
    h = _agg(h, av, lv, uv) + b2_ref[...]
    mu_ref[...] = h[:, :GNN_ACT]
    ls = jnp.tanh(h[:, GNN_ACT:])
    ls = LOG_STD_MIN + 0.5 * (LOG_STD_MAX - LOG_STD_MIN) * (ls + 1.0)
    std_ref[...] = jnp.exp(ls)


@functools.partial(jax.jit, static_argnames=())
def kernel(obs, W0, b0, W1, b1, W2, b2):
    bs = obs.shape[0]
    rows = bs * NUM_NODES
    block_rows = BATCH_BLOCK * NUM_NODES
    grid = (bs // BATCH_BLOCK,)

    x = obs.reshape(rows, GNN_OBS)

    const = lambda shape: pl.BlockSpec(shape, lambda i: (0, 0))
    mu2, std2 = pl.pallas_call(
        _fused_kernel,
        grid=grid,
        in_specs=[
            pl.BlockSpec((block_rows, GNN_OBS), lambda i: (i, 0)),
            const((GNN_OBS, HIDDEN)),
            const((1, HIDDEN)),
            const((HIDDEN, HIDDEN)),
            const((1, HIDDEN)),
            const((HIDDEN, 2 * GNN_ACT)),
            const((1, 2 * GNN_ACT)),
        ],
        out_specs=[
            pl.BlockSpec((block_rows, GNN_ACT), lambda i: (i, 0)),
            pl.BlockSpec((block_rows, GNN_ACT), lambda i: (i, 0)),
        ],
        out_shape=[
            jax.ShapeDtypeStruct((rows, GNN_ACT), jnp.float32),
            jax.ShapeDtypeStruct((rows, GNN_ACT), jnp.float32),
        ],
    )(x, W0, b0.reshape(1, HIDDEN), W1, b1.reshape(1, HIDDEN),
      W2, b2.reshape(1, 2 * GNN_ACT))

    mu = mu2.reshape(bs, NUM_NODES * GNN_ACT)
    std = std2.reshape(bs, NUM_NODES * GNN_ACT)
    return (mu, std)
